# baseline (speedup 1.0000x reference)
import functools

import jax
import jax.numpy as jnp
from jax import lax
from jax.experimental import pallas as pl
from jax.experimental.pallas import tpu as pltpu
from jax.experimental.pallas import tpu_sc as plsc

N = 262144
NB = 100
EPS = 1e-10
NC = 2
NS = 16
L = 16
NW = NC * NS
SHARD = N // NS
HALF = SHARD // NC
M = 3
NROW = 4 * M
CW = 144
MROW = NROW * CW
R4 = 40


def _sc_moments(q, p):
    mesh = plsc.VectorSubcoreMesh(core_axis_name="c", subcore_axis_name="s")

    @functools.partial(
        pl.kernel,
        out_type=[
            jax.ShapeDtypeStruct((NW * MROW,), jnp.float32),
            jax.ShapeDtypeStruct((NW * MROW,), jnp.float32),
            jax.ShapeDtypeStruct((L,), jnp.float32),
        ],
        mesh=mesh,
        compiler_params=pltpu.CompilerParams(needs_layout_passes=False),
        scratch_types=[
            pltpu.VMEM((SHARD,), jnp.float32),
            pltpu.VMEM((SHARD,), jnp.float32),
            pltpu.VMEM((MROW,), jnp.float32),
            pltpu.VMEM((MROW,), jnp.float32),
            pltpu.VMEM((2 * L,), jnp.float32),
            pltpu.VMEM((NS * 2 * L,), jnp.float32),
            pltpu.VMEM_SHARED((NS * 2 * L,), jnp.float32),
        ],
    )
    def mom_kernel(q_hbm, p_hbm, oq_hbm, op_hbm, omm_hbm, xq_v, xp_v, mq_v,
                   mp_v, mm_v, allmm_v, shared_mm):
        c = lax.axis_index("c")
        s = lax.axis_index("s")

        pltpu.sync_copy(q_hbm.at[pl.ds(s * SHARD, SHARD)], xq_v)
        pltpu.sync_copy(p_hbm.at[pl.ds(s * SHARD, SHARD)], xp_v)

        def mm_body(i, carry):
            mn, mx = carry
            for j in range(8):
                a = xq_v[pl.ds((i * 8 + j) * L, L)]
                b = xp_v[pl.ds((i * 8 + j) * L, L)]
                mn = jnp.minimum(mn, jnp.minimum(a, b))
                mx = jnp.maximum(mx, jnp.maximum(a, b))
            return (mn, mx)

        first = xq_v[pl.ds(0, L)]
        mn, mx = lax.fori_loop(0, SHARD // L // 8, mm_body, (first, first))
        mm_v[pl.ds(0, L)] = mn
        mm_v[pl.ds(L, L)] = mx

        pltpu.sync_copy(mm_v, shared_mm.at[pl.ds(s * 2 * L, 2 * L)])
        plsc.subcore_barrier()
        pltpu.sync_copy(shared_mm, allmm_v)
        amn = allmm_v[pl.ds(0, L)]
        amx = allmm_v[pl.ds(L, L)]
        for t in range(1, NS):
            amn = jnp.minimum(amn, allmm_v[pl.ds(t * 2 * L, L)])
            amx = jnp.maximum(amx, allmm_v[pl.ds(t * 2 * L + L, L)])
        lane = lax.iota(jnp.int32, L)
        for sh in (8, 4, 2, 1):
            perm = (lane + sh) & (L - 1)
            amn = jnp.minimum(amn, amn.at[perm].get(mode="promise_in_bounds"))
            amx = jnp.maximum(amx, amx.at[perm].get(mode="promise_in_bounds"))

        wid = c * NS + s

        @pl.when(wid == 0)
        def _():
            mm_v[pl.ds(0, L)] = jnp.where(lane < 8, amn, amx)
            pltpu.sync_copy(mm_v.at[pl.ds(0, L)], omm_hbm)

        gmn = amn
        rng = amx - amn
        invd4 = (4.0 * (NB - 1.0)) / rng
        dh4 = rng * (10.0 / (4.0 * (NB - 1.0)))

        zeros = jnp.zeros((L,), jnp.float32)

        def z_body(i, _):
            mq_v[pl.ds(i * L, L)] = zeros
            mp_v[pl.ds(i * L, L)] = zeros
            return 0

        lax.fori_loop(0, MROW // L, z_body, 0)

        base = c * HALF

        def chunk(i, _):
            for j in range(4):
                for x_v, m_v in ((xq_v, mq_v), (xp_v, mp_v)):
                    x = x_v[pl.ds(base + (i * 4 + j) * L, L)]
                    u4 = (x - gmn) * invd4
                    jv = (u4 + 0.5).astype(jnp.int32)
                    t = (u4 - jv.astype(jnp.float32)) * dh4
                    idx = ((jv & 3) * CW + (jv >> 2)) + 16
                    t2 = t * t
                    g = (t2 * 0.125 - 0.5) * t2 + 1.0
                    plsc.addupdate_scatter(m_v, [idx], g)
                    v1 = g * t
                    plsc.addupdate_scatter(m_v, [idx + 4 * CW], v1)
                    v2 = v1 * (t * 0.5)
                    plsc.addupdate_scatter(m_v, [idx + 8 * CW], v2)
            return 0

        lax.fori_loop(0, HALF // L // 4, chunk, 0)

        pltpu.sync_copy(mq_v, oq_hbm.at[pl.ds(wid * MROW, MROW)])
        pltpu.sync_copy(mp_v, op_hbm.at[pl.ds(wid * MROW, MROW)])

    return mom_kernel(q, p)


def _tc_jsd(tq, tp, mm):
    def body(tq_ref, tp_ref, mm_ref, o_ref):
        sq = jnp.sum(tq_ref[...], axis=0)
        sp = jnp.sum(tp_ref[...], axis=0)
        gmn = mm_ref[0, 0]
        gmx = mm_ref[0, 8]
        dq = (gmx - gmn) * (10.0 / (4.0 * (NB - 1.0)))

        outq = jnp.zeros((1, 104), jnp.float32)
        outp = jnp.zeros((1, 104), jnp.float32)
        for r in range(-R4, R4 + 1):
            qr, pr = divmod(r, 4)
            rd = r * dq
            e = jnp.exp(-0.5 * (rd * rd))
            b = -rd
            coef = e
            for m in range(M):
                row = m * 4 + pr
                outq = outq + coef * sq[row:row + 1, 16 + qr:120 + qr]
                outp = outp + coef * sp[row:row + 1, 16 + qr:120 + qr]
                coef = coef * b
        colid = lax.broadcasted_iota(jnp.int32, (1, 104), 1)
        mask = colid < NB
        pdfq = jnp.where(mask, outq * (1.0 / N), 0.0)
        pdfp = jnp.where(mask, outp * (1.0 / N), 0.0)
        qh = pdfq / (jnp.sum(pdfq) + EPS)
        ph = pdfp / (jnp.sum(pdfp) + EPS)
        mh = 0.5 * (ph + qh)
        qh = jnp.maximum(qh, 1e-45)
        ph = jnp.maximum(ph, 1e-45)
        mh = jnp.maximum(mh, 1e-45)
        lp = jnp.log(ph)
        lq = jnp.log(qh)
        lm = jnp.log(mh)
        tsum = jnp.exp(lp) * (lp - lm) + jnp.exp(lq) * (lq - lm)
        o_ref[...] = 0.5 * jnp.sum(jnp.where(mask, tsum, 0.0), keepdims=True)

    return pl.pallas_call(
        body,
        out_shape=jax.ShapeDtypeStruct((1, 1), jnp.float32),
    )(tq, tp, mm)


def kernel(q, p):
    oq, op, omm = _sc_moments(q, p)
    return _tc_jsd(oq.reshape(NW, NROW, CW), op.reshape(NW, NROW, CW),
                   omm.reshape(1, L))[0, 0]


# speedup vs baseline: 1.1112x; 1.1112x over previous
import functools

import jax
import jax.numpy as jnp
from jax import lax
from jax.experimental import pallas as pl
from jax.experimental.pallas import tpu as pltpu
from jax.experimental.pallas import tpu_sc as plsc

N = 262144
NB = 100
EPS = 1e-10
NC = 2
NS = 16
L = 16
NW = NC * NS
SHARD = N // NS
HALF = SHARD // NC
M = 3
NROW = 4 * M
CW = 144
MROW = NROW * CW
R4 = 40


def _sc_moments(q, p):
    mesh = plsc.VectorSubcoreMesh(core_axis_name="c", subcore_axis_name="s")

    @functools.partial(
        pl.kernel,
        out_type=[
            jax.ShapeDtypeStruct((NW * MROW,), jnp.float32),
            jax.ShapeDtypeStruct((NW * MROW,), jnp.float32),
            jax.ShapeDtypeStruct((L,), jnp.float32),
        ],
        mesh=mesh,
        compiler_params=pltpu.CompilerParams(needs_layout_passes=False),
        scratch_types=[
            pltpu.VMEM((SHARD,), jnp.float32),
            pltpu.VMEM((SHARD,), jnp.float32),
            pltpu.VMEM((MROW,), jnp.float32),
            pltpu.VMEM((MROW,), jnp.float32),
            pltpu.VMEM((2 * L,), jnp.float32),
            pltpu.VMEM((NS * 2 * L,), jnp.float32),
            pltpu.VMEM_SHARED((NS * 2 * L,), jnp.float32),
        ],
    )
    def mom_kernel(q_hbm, p_hbm, oq_hbm, op_hbm, omm_hbm, xq_v, xp_v, mq_v,
                   mp_v, mm_v, allmm_v, shared_mm):
        c = lax.axis_index("c")
        s = lax.axis_index("s")

        pltpu.sync_copy(q_hbm.at[pl.ds(s * SHARD, SHARD)], xq_v)
        pltpu.sync_copy(p_hbm.at[pl.ds(s * SHARD, SHARD)], xp_v)

        def mm_body(i, carry):
            mn, mx = carry
            for j in range(4):
                a = xq_v[pl.ds((i * 4 + j) * L, L)]
                b = xp_v[pl.ds((i * 4 + j) * L, L)]
                mn = jnp.minimum(mn, jnp.minimum(a, b))
                mx = jnp.maximum(mx, jnp.maximum(a, b))
            return (mn, mx)

        first = xq_v[pl.ds(0, L)]
        mn, mx = lax.fori_loop(0, SHARD // L // 4, mm_body, (first, first))
        mm_v[pl.ds(0, L)] = mn
        mm_v[pl.ds(L, L)] = mx

        pltpu.sync_copy(mm_v, shared_mm.at[pl.ds(s * 2 * L, 2 * L)])
        plsc.subcore_barrier()
        pltpu.sync_copy(shared_mm, allmm_v)
        amn = allmm_v[pl.ds(0, L)]
        amx = allmm_v[pl.ds(L, L)]
        for t in range(1, NS):
            amn = jnp.minimum(amn, allmm_v[pl.ds(t * 2 * L, L)])
            amx = jnp.maximum(amx, allmm_v[pl.ds(t * 2 * L + L, L)])
        lane = lax.iota(jnp.int32, L)
        for sh in (8, 4, 2, 1):
            perm = (lane + sh) & (L - 1)
            amn = jnp.minimum(amn, amn.at[perm].get(mode="promise_in_bounds"))
            amx = jnp.maximum(amx, amx.at[perm].get(mode="promise_in_bounds"))

        wid = c * NS + s

        @pl.when(wid == 0)
        def _():
            mm_v[pl.ds(0, L)] = jnp.where(lane < 8, amn, amx)
            pltpu.sync_copy(mm_v.at[pl.ds(0, L)], omm_hbm)

        gmn = amn
        rng = amx - amn
        invd4 = (4.0 * (NB - 1.0)) / rng
        dh4 = rng * (10.0 / (4.0 * (NB - 1.0)))

        zeros = jnp.zeros((L,), jnp.float32)

        def z_body(i, _):
            mq_v[pl.ds(i * L, L)] = zeros
            mp_v[pl.ds(i * L, L)] = zeros
            return 0

        lax.fori_loop(0, MROW // L, z_body, 0)

        base = c * HALF
        ones = jnp.ones((L,), jnp.float32)

        def chunk(i, _):
            for j in range(2):
                for x_v, m_v in ((xq_v, mq_v), (xp_v, mp_v)):
                    x = x_v[pl.ds(base + (i * 2 + j) * L, L)]
                    u4 = (x - gmn) * invd4
                    jv = (u4 + 0.5).astype(jnp.int32)
                    t = (u4 - jv.astype(jnp.float32)) * dh4
                    idx = ((jv & 3) * CW + (jv >> 2)) + 16
                    plsc.addupdate_scatter(m_v, [idx], ones)
                    plsc.addupdate_scatter(m_v, [idx + 4 * CW], t)
                    plsc.addupdate_scatter(m_v, [idx + 8 * CW], t * t * 0.5)
            return 0

        lax.fori_loop(0, HALF // L // 2, chunk, 0)

        pltpu.sync_copy(mq_v, oq_hbm.at[pl.ds(wid * MROW, MROW)])
        pltpu.sync_copy(mp_v, op_hbm.at[pl.ds(wid * MROW, MROW)])

    return mom_kernel(q, p)


def _tc_jsd(tq, tp, mm):
    def body(tq_ref, tp_ref, mm_ref, o_ref):
        sq = jnp.sum(tq_ref[...], axis=0)
        sp = jnp.sum(tp_ref[...], axis=0)
        gmn = mm_ref[0, 0]
        gmx = mm_ref[0, 8]
        dq = (gmx - gmn) * (10.0 / (4.0 * (NB - 1.0)))

        outq = jnp.zeros((1, 104), jnp.float32)
        outp = jnp.zeros((1, 104), jnp.float32)
        for r in range(-R4, R4 + 1):
            qr, pr = divmod(r, 4)
            rd = r * dq
            e = jnp.exp(-0.5 * (rd * rd))
            coefs = (e, -e * rd, e * (rd * rd - 1.0))
            for m in range(M):
                row = m * 4 + pr
                outq = outq + coefs[m] * sq[row:row + 1, 16 + qr:120 + qr]
                outp = outp + coefs[m] * sp[row:row + 1, 16 + qr:120 + qr]
        colid = lax.broadcasted_iota(jnp.int32, (1, 104), 1)
        mask = colid < NB
        pdfq = jnp.where(mask, outq * (1.0 / N), 0.0)
        pdfp = jnp.where(mask, outp * (1.0 / N), 0.0)
        qh = pdfq / (jnp.sum(pdfq) + EPS)
        ph = pdfp / (jnp.sum(pdfp) + EPS)
        mh = 0.5 * (ph + qh)
        qh = jnp.maximum(qh, 1e-45)
        ph = jnp.maximum(ph, 1e-45)
        mh = jnp.maximum(mh, 1e-45)
        lp = jnp.log(ph)
        lq = jnp.log(qh)
        lm = jnp.log(mh)
        tsum = jnp.exp(lp) * (lp - lm) + jnp.exp(lq) * (lq - lm)
        o_ref[...] = 0.5 * jnp.sum(jnp.where(mask, tsum, 0.0), keepdims=True)

    return pl.pallas_call(
        body,
        out_shape=jax.ShapeDtypeStruct((1, 1), jnp.float32),
    )(tq, tp, mm)


def kernel(q, p):
    oq, op, omm = _sc_moments(q, p)
    return _tc_jsd(oq.reshape(NW, NROW, CW), op.reshape(NW, NROW, CW),
                   omm.reshape(1, L))[0, 0]
